# Initial kernel scaffold; baseline (speedup 1.0000x reference)
import functools

import jax
import jax.numpy as jnp
from jax import lax
from jax.experimental import pallas as pl
from jax.experimental.pallas import tpu as pltpu
from jax.experimental.pallas import tpu_sc as plsc

N = 10000
K = 32
F = 128
NUP = 2500

NPAD = 10240
C = 144
EB = 128
NB = NPAD * K // EB
NW = 32
BPW = NB // NW
RPS = NPAD // 16
NUPPAD = 2560
SPW = NUPPAD // NW

_mesh = plsc.VectorSubcoreMesh(core_axis_name="c", subcore_axis_name="s")


_CBR = 16


def _contrib_body(w_ref, f_ref, o_ref):
    w = w_ref[...]
    f = f_ref[...]
    outer = (w[:, :, None] * f[:, None, :]).reshape(_CBR, EB, F)
    wcol = w.reshape(_CBR, EB, 1)
    pad = jnp.zeros((_CBR, EB, C - F - 1), jnp.float32)
    o_ref[...] = jnp.concatenate([outer, wcol, pad], axis=2)


def _build_contrib(w_pad, f_pad):
    return pl.pallas_call(
        _contrib_body,
        grid=(NB // _CBR,),
        in_specs=[
            pl.BlockSpec((_CBR * 4, K), lambda i: (i, 0)),
            pl.BlockSpec((_CBR * 4, F), lambda i: (i, 0)),
        ],
        out_specs=pl.BlockSpec((_CBR, EB, C), lambda i: (i, 0, 0)),
        out_shape=jax.ShapeDtypeStruct((NB, EB, C), jnp.float32),
    )(w_pad, f_pad)


@functools.partial(
    pl.kernel,
    out_type=[
        jax.ShapeDtypeStruct((NPAD, C), jnp.float32),
        jax.ShapeDtypeStruct((NPAD, C), jnp.float32),
    ],
    mesh=_mesh,
    scratch_types=[
        pltpu.VMEM_SHARED((NPAD, C), jnp.float32),
        pltpu.VMEM((EB, C), jnp.float32),
        pltpu.VMEM((1, EB), jnp.int32),
    ],
)
def _scatter_kernel(contrib_hbm, nidx_hbm, zeros_hbm, p0_hbm, p1_hbm,
                    acc, cbuf, ibuf):
    c = lax.axis_index("c")
    s = lax.axis_index("s")
    wid = s * 2 + c

    pltpu.sync_copy(zeros_hbm, acc.at[pl.ds(s * RPS, RPS)])
    plsc.subcore_barrier()

    @pl.loop(0, BPW)
    def _(j):
        b = wid * BPW + j
        pltpu.sync_copy(contrib_hbm.at[b], cbuf)
        pltpu.sync_copy(nidx_hbm.at[b], ibuf.at[0])
        pltpu.sync_copy(cbuf, acc.at[ibuf.at[0]], add=True)

    plsc.subcore_barrier()

    @pl.when(c == 0)
    def _():
        pltpu.sync_copy(acc.at[pl.ds(s * RPS, RPS)],
                        p0_hbm.at[pl.ds(s * RPS, RPS)])

    @pl.when(c == 1)
    def _():
        pltpu.sync_copy(acc.at[pl.ds(s * RPS, RPS)],
                        p1_hbm.at[pl.ds(s * RPS, RPS)])


@functools.partial(
    pl.kernel,
    out_type=[
        jax.ShapeDtypeStruct((NUPPAD, C), jnp.float32),
        jax.ShapeDtypeStruct((NUPPAD, C), jnp.float32),
    ],
    mesh=_mesh,
    scratch_types=[
        pltpu.VMEM((SPW,), jnp.int32),
        pltpu.VMEM((SPW, C), jnp.float32),
        pltpu.VMEM((SPW, C), jnp.float32),
    ],
)
def _gather_kernel(p0_hbm, p1_hbm, sel_hbm, g0_hbm, g1_hbm, idxb, b0, b1):
    c = lax.axis_index("c")
    s = lax.axis_index("s")
    base = (s * 2 + c) * SPW
    pltpu.sync_copy(sel_hbm.at[pl.ds(base, SPW)], idxb)
    pltpu.sync_copy(p0_hbm.at[idxb], b0)
    pltpu.sync_copy(p1_hbm.at[idxb], b1)
    pltpu.sync_copy(b0, g0_hbm.at[pl.ds(base, SPW)])
    pltpu.sync_copy(b1, g1_hbm.at[pl.ds(base, SPW)])


def _norm_body(a_ref, b_ref, o_ref):
    h = a_ref[...] + b_ref[...]
    wsum = jnp.maximum(h[:, F:F + 1], 0.0)
    wsum = jnp.where(wsum > 0.0, wsum, 0.001)
    o_ref[...] = h[:, :F] / wsum


def _normalize(g0, g1):
    return pl.pallas_call(
        _norm_body,
        out_shape=jax.ShapeDtypeStruct((NUPPAD, F), jnp.float32),
    )(g0, g1)


def kernel(features, nidx_down, weights_down, sel_idx_up):
    f_pad = jnp.pad(features, ((0, NPAD - N), (0, 0)))
    w_pad = jnp.pad(weights_down, ((0, NPAD - N), (0, 0)))
    n_pad = jnp.pad(nidx_down, ((0, NPAD - N), (0, 0)))
    nidx2d = n_pad.reshape(NB, EB)
    selpad = jnp.pad(sel_idx_up[:, 0], (0, NUPPAD - NUP))
    zeros = jnp.zeros((RPS, C), jnp.float32)

    contrib = _build_contrib(w_pad, f_pad)
    p0, p1 = _scatter_kernel(contrib, nidx2d, zeros)
    g0, g1 = _gather_kernel(p0, p1, selpad)
    out = _normalize(g0, g1)
    return out[:NUP]


# speedup vs baseline: 3.6263x; 3.6263x over previous
import dataclasses
import functools

import jax
import jax.numpy as jnp
from jax import lax
from jax.experimental import pallas as pl
from jax.experimental.pallas import tpu as pltpu
from jax.experimental.pallas import tpu_sc as plsc

N = 10000
K = 32
F = 128
NUP = 2500

NPAD = 10240
EB = 128
NB = NPAD * K // EB
NW = 32
NS = 16
BPW = NB // NW
RPS = NPAD // NS
NUPPAD = 2560
SPW = NUPPAD // NW
L = 16


@functools.cache
def _mesh():
    return plsc.VectorSubcoreMesh(core_axis_name="c", subcore_axis_name="s")


def _sc_params():
    cp = pltpu.CompilerParams()
    if "needs_layout_passes" in pltpu.CompilerParams.__dataclass_fields__:
        cp = dataclasses.replace(cp, needs_layout_passes=False)
    return cp


_CBR = 16


def _contrib_body(w_ref, f_ref, o_ref):
    w = w_ref[...]
    f = f_ref[...]
    o_ref[...] = (w[:, :, None] * f[:, None, :]).reshape(_CBR, EB, F)


def _build_contrib(w_pad, f_pad):
    return pl.pallas_call(
        _contrib_body,
        grid=(NB // _CBR,),
        in_specs=[
            pl.BlockSpec((_CBR * 4, K), lambda i: (i, 0)),
            pl.BlockSpec((_CBR * 4, F), lambda i: (i, 0)),
        ],
        out_specs=pl.BlockSpec((_CBR, EB, F), lambda i: (i, 0, 0)),
        out_shape=jax.ShapeDtypeStruct((NB, EB, F), jnp.float32),
    )(w_pad, f_pad)


@functools.cache
def _scatter_kernel():
    return pl.kernel(
        _scatter_body,
        out_type=[
            jax.ShapeDtypeStruct((NPAD, F), jnp.float32),
            jax.ShapeDtypeStruct((NPAD, F), jnp.float32),
            jax.ShapeDtypeStruct((NPAD,), jnp.float32),
            jax.ShapeDtypeStruct((NPAD,), jnp.float32),
        ],
        mesh=_mesh(),
        scratch_types=[
            pltpu.VMEM_SHARED((NPAD, F), jnp.float32),
            pltpu.VMEM_SHARED((NS, NPAD), jnp.float32),
            pltpu.VMEM((EB, F), jnp.float32),
            pltpu.VMEM((1, EB), jnp.int32),
            pltpu.VMEM((1, EB), jnp.float32),
            pltpu.VMEM((NPAD,), jnp.float32),
            pltpu.VMEM((NS, RPS), jnp.float32),
            pltpu.VMEM((RPS,), jnp.float32),
        ],
        compiler_params=_sc_params(),
    )


def _scatter_body(contrib_hbm, nidx_hbm, w_hbm, zeros_hbm, zrow_hbm,
                  p0_hbm, p1_hbm, ws0_hbm, ws1_hbm,
                  acc, ws_stage, cbuf, ibuf, wbuf, wsacc, wred, wout):
    c = lax.axis_index("c")
    s = lax.axis_index("s")
    wid = s * 2 + c

    pltpu.sync_copy(zeros_hbm, acc.at[pl.ds(s * RPS, RPS)])
    pltpu.sync_copy(zrow_hbm, wsacc)
    plsc.subcore_barrier()

    @pl.loop(0, BPW)
    def _(j):
        b = wid * BPW + j
        pltpu.sync_copy(contrib_hbm.at[b], cbuf)
        pltpu.sync_copy(nidx_hbm.at[b], ibuf.at[0])
        pltpu.sync_copy(w_hbm.at[b], wbuf.at[0])
        pltpu.sync_copy(cbuf, acc.at[ibuf.at[0]], add=True)
        for g in range(EB // L):
            dst = ibuf[0, pl.ds(g * L, L)]
            wv = wbuf[0, pl.ds(g * L, L)]
            plsc.addupdate_scatter(wsacc, [dst], wv)

    pltpu.sync_copy(wsacc, ws_stage.at[s])
    plsc.subcore_barrier()

    @pl.when(c == 0)
    def _():
        pltpu.sync_copy(acc.at[pl.ds(s * RPS, RPS)],
                        p0_hbm.at[pl.ds(s * RPS, RPS)])

    @pl.when(c == 1)
    def _():
        pltpu.sync_copy(acc.at[pl.ds(s * RPS, RPS)],
                        p1_hbm.at[pl.ds(s * RPS, RPS)])

    @pl.loop(0, NS)
    def _(r):
        pltpu.sync_copy(ws_stage.at[r, pl.ds(s * RPS, RPS)],
                        wred.at[r, pl.ds(0, RPS)])

    @pl.loop(0, RPS // L)
    def _(v):
        tot = wred[0, pl.ds(v * L, L)]
        for r in range(1, NS):
            tot = tot + wred[r, pl.ds(v * L, L)]
        wout[pl.ds(v * L, L)] = tot

    @pl.when(c == 0)
    def _():
        pltpu.sync_copy(wout, ws0_hbm.at[pl.ds(s * RPS, RPS)])

    @pl.when(c == 1)
    def _():
        pltpu.sync_copy(wout, ws1_hbm.at[pl.ds(s * RPS, RPS)])


@functools.cache
def _gather_kernel():
    return pl.kernel(
        _gather_body,
        out_type=[
            jax.ShapeDtypeStruct((NUPPAD, F), jnp.float32),
            jax.ShapeDtypeStruct((NUPPAD, F), jnp.float32),
            jax.ShapeDtypeStruct((NUPPAD,), jnp.float32),
        ],
        mesh=_mesh(),
        scratch_types=[
            pltpu.VMEM((SPW,), jnp.int32),
            pltpu.VMEM((SPW, F), jnp.float32),
            pltpu.VMEM((SPW, F), jnp.float32),
            pltpu.VMEM((NPAD,), jnp.float32),
            pltpu.VMEM((NPAD,), jnp.float32),
            pltpu.VMEM((SPW,), jnp.float32),
        ],
        compiler_params=_sc_params(),
    )


def _gather_body(p0_hbm, p1_hbm, ws0_hbm, ws1_hbm, sel_hbm,
                 g0_hbm, g1_hbm, wsel_hbm, idxb, b0, b1, wt0, wt1, wsb):
    c = lax.axis_index("c")
    s = lax.axis_index("s")
    base = (s * 2 + c) * SPW
    pltpu.sync_copy(sel_hbm.at[pl.ds(base, SPW)], idxb)
    pltpu.sync_copy(p0_hbm.at[idxb], b0)
    pltpu.sync_copy(p1_hbm.at[idxb], b1)
    pltpu.sync_copy(ws0_hbm, wt0)
    pltpu.sync_copy(ws1_hbm, wt1)

    @pl.loop(0, NPAD // L)
    def _(v):
        wt0[pl.ds(v * L, L)] = wt0[pl.ds(v * L, L)] + wt1[pl.ds(v * L, L)]

    @pl.loop(0, SPW // L)
    def _(g):
        sv = idxb[pl.ds(g * L, L)]
        wsb[pl.ds(g * L, L)] = plsc.load_gather(wt0, [sv])

    pltpu.sync_copy(b0, g0_hbm.at[pl.ds(base, SPW)])
    pltpu.sync_copy(b1, g1_hbm.at[pl.ds(base, SPW)])
    pltpu.sync_copy(wsb, wsel_hbm.at[pl.ds(base, SPW)])


def _norm_body(a_ref, b_ref, w_ref, o_ref):
    h = a_ref[...] + b_ref[...]
    wsum = jnp.maximum(w_ref[...], 0.0)
    wsum = jnp.where(wsum > 0.0, wsum, 0.001)
    o_ref[...] = h / wsum


def _normalize(g0, g1, wsel):
    return pl.pallas_call(
        _norm_body,
        out_shape=jax.ShapeDtypeStruct((NUPPAD, F), jnp.float32),
    )(g0, g1, wsel)


def kernel(features, nidx_down, weights_down, sel_idx_up):
    f_pad = jnp.pad(features, ((0, NPAD - N), (0, 0)))
    w_pad = jnp.pad(weights_down, ((0, NPAD - N), (0, 0)))
    n_pad = jnp.pad(nidx_down, ((0, NPAD - N), (0, 0)))
    nidx2d = n_pad.reshape(NB, EB)
    w2d = w_pad.reshape(NB, EB)
    selpad = jnp.pad(sel_idx_up[:, 0], (0, NUPPAD - NUP))
    zeros = jnp.zeros((RPS, F), jnp.float32)
    zrow = jnp.zeros((NPAD,), jnp.float32)

    contrib = _build_contrib(w_pad, f_pad)
    p0, p1, ws0, ws1 = _scatter_kernel()(contrib, nidx2d, w2d, zeros, zrow)
    g0, g1, wsel = _gather_kernel()(p0, p1, ws0, ws1, selpad)
    out = _normalize(g0, g1, wsel.reshape(NUPPAD, 1))
    return out[:NUP]


# speedup vs baseline: 4.5137x; 1.2447x over previous
import dataclasses
import functools

import jax
import jax.numpy as jnp
from jax import lax
from jax.experimental import pallas as pl
from jax.experimental.pallas import tpu as pltpu
from jax.experimental.pallas import tpu_sc as plsc

N = 10000
K = 32
F = 128
NUP = 2500

NPAD = 10240
EB = 128
NB = NPAD * K // EB
NW = 32
NS = 16
BPW = NB // NW
RPS = NPAD // NS
NUPPAD = 2560
SPW = NUPPAD // NW
L = 16


@functools.cache
def _mesh():
    return plsc.VectorSubcoreMesh(core_axis_name="c", subcore_axis_name="s")


def _sc_params():
    cp = pltpu.CompilerParams()
    if "needs_layout_passes" in pltpu.CompilerParams.__dataclass_fields__:
        cp = dataclasses.replace(cp, needs_layout_passes=False)
    return cp


_CBR = 16


def _contrib_body(w_ref, f_ref, o_ref):
    w = w_ref[...]
    f = f_ref[...]
    o_ref[...] = (w[:, :, None] * f[:, None, :]).reshape(_CBR, EB, F)


def _build_contrib(w_pad, f_pad):
    return pl.pallas_call(
        _contrib_body,
        grid=(NB // _CBR,),
        in_specs=[
            pl.BlockSpec((_CBR * 4, K), lambda i: (i, 0)),
            pl.BlockSpec((_CBR * 4, F), lambda i: (i, 0)),
        ],
        out_specs=pl.BlockSpec((_CBR, EB, F), lambda i: (i, 0, 0)),
        out_shape=jax.ShapeDtypeStruct((NB, EB, F), jnp.float32),
    )(w_pad, f_pad)


@functools.cache
def _scatter_kernel():
    return pl.kernel(
        _scatter_body,
        out_type=[
            jax.ShapeDtypeStruct((NPAD, F), jnp.float32),
            jax.ShapeDtypeStruct((NPAD, F), jnp.float32),
            jax.ShapeDtypeStruct((2, NS, NPAD), jnp.float32),
        ],
        mesh=_mesh(),
        scratch_types=[
            pltpu.VMEM_SHARED((NPAD, F), jnp.float32),
            pltpu.VMEM((2, EB, F), jnp.float32),
            pltpu.VMEM((2, EB), jnp.int32),
            pltpu.VMEM((2, EB), jnp.float32),
            pltpu.VMEM((NPAD,), jnp.float32),
            pltpu.SemaphoreType.DMA((2,)),
            pltpu.SemaphoreType.DMA((2,)),
            pltpu.SemaphoreType.DMA((2,)),
            pltpu.SemaphoreType.DMA((2,)),
        ],
        compiler_params=_sc_params(),
    )


def _scatter_body(contrib_hbm, nidx_hbm, w_hbm, zeros_hbm, zrow_hbm,
                  p0_hbm, p1_hbm, wsparts_hbm,
                  acc, cbuf, ibuf, wbuf, wsacc, csem, isem, wsem, ssem):
    c = lax.axis_index("c")
    s = lax.axis_index("s")
    wid = s * 2 + c
    base = wid * BPW

    def _contrib_dma(j, b):
        return pltpu.make_async_copy(contrib_hbm.at[base + j], cbuf.at[b],
                                     csem.at[b])

    def _idx_dma(j, b):
        return pltpu.make_async_copy(nidx_hbm.at[base + j], ibuf.at[b],
                                     isem.at[b])

    def _w_dma(j, b):
        return pltpu.make_async_copy(w_hbm.at[base + j], wbuf.at[b],
                                     wsem.at[b])

    for b in range(2):
        _contrib_dma(b, b).start()
        _idx_dma(b, b).start()
        _w_dma(b, b).start()
    pltpu.sync_copy(zeros_hbm, acc.at[pl.ds(s * RPS, RPS)])
    pltpu.sync_copy(zrow_hbm, wsacc)
    plsc.subcore_barrier()

    @pl.loop(0, BPW, step=2)
    def _(j):
        streams = []
        for b in range(2):
            jj = j + b
            _idx_dma(jj, b).wait()
            _w_dma(jj, b).wait()
            for g in range(EB // L):
                dst = ibuf[b, pl.ds(g * L, L)]
                wv = wbuf[b, pl.ds(g * L, L)]
                plsc.addupdate_scatter(wsacc, [dst], wv)
            _contrib_dma(jj, b).wait()
            streams.append(pltpu.async_copy(cbuf.at[b], acc.at[ibuf.at[b]],
                                            ssem.at[b], add=True))
        for b in range(2):
            streams[b].wait()

            @pl.when(j + 2 + b < BPW)
            def _():
                _contrib_dma(j + 2 + b, b).start()
                _idx_dma(j + 2 + b, b).start()
                _w_dma(j + 2 + b, b).start()

    plsc.subcore_barrier()

    @pl.when(c == 0)
    def _():
        pltpu.sync_copy(acc.at[pl.ds(s * RPS, RPS)],
                        p0_hbm.at[pl.ds(s * RPS, RPS)])

    @pl.when(c == 1)
    def _():
        pltpu.sync_copy(acc.at[pl.ds(s * RPS, RPS)],
                        p1_hbm.at[pl.ds(s * RPS, RPS)])

    pltpu.sync_copy(wsacc, wsparts_hbm.at[c, s])


@functools.cache
def _gather_kernel():
    return pl.kernel(
        _gather_body,
        out_type=[
            jax.ShapeDtypeStruct((NUPPAD, F), jnp.float32),
            jax.ShapeDtypeStruct((NUPPAD, F), jnp.float32),
            jax.ShapeDtypeStruct((NUPPAD,), jnp.float32),
        ],
        mesh=_mesh(),
        scratch_types=[
            pltpu.VMEM_SHARED((NPAD,), jnp.float32),
            pltpu.VMEM((SPW,), jnp.int32),
            pltpu.VMEM((SPW, F), jnp.float32),
            pltpu.VMEM((SPW, F), jnp.float32),
            pltpu.VMEM((NPAD,), jnp.float32),
            pltpu.VMEM((RPS,), jnp.float32),
            pltpu.VMEM((RPS,), jnp.float32),
            pltpu.VMEM((SPW,), jnp.float32),
        ],
        compiler_params=_sc_params(),
    )


def _gather_body(p0_hbm, p1_hbm, wsparts_hbm, sel_hbm,
                 g0_hbm, g1_hbm, wsel_hbm,
                 wstot_sh, idxb, b0, b1, wt0, wsl, wtmp, wsb):
    c = lax.axis_index("c")
    s = lax.axis_index("s")
    base = (s * 2 + c) * SPW
    pltpu.sync_copy(sel_hbm.at[pl.ds(base, SPW)], idxb)
    pltpu.sync_copy(p0_hbm.at[idxb], b0)
    pltpu.sync_copy(p1_hbm.at[idxb], b1)

    @pl.loop(0, RPS // L)
    def _(v):
        wsl[pl.ds(v * L, L)] = jnp.zeros((L,), jnp.float32)

    for c2 in range(2):
        @pl.loop(0, NS)
        def _(r):
            pltpu.sync_copy(wsparts_hbm.at[c2, r, pl.ds(s * RPS, RPS)], wtmp)

            @pl.loop(0, RPS // L)
            def _(v):
                wsl[pl.ds(v * L, L)] = wsl[pl.ds(v * L, L)] + wtmp[pl.ds(v * L, L)]

    pltpu.sync_copy(wsl, wstot_sh.at[pl.ds(s * RPS, RPS)])
    plsc.subcore_barrier()
    pltpu.sync_copy(wstot_sh, wt0)

    @pl.loop(0, SPW // L)
    def _(g):
        sv = idxb[pl.ds(g * L, L)]
        wsb[pl.ds(g * L, L)] = plsc.load_gather(wt0, [sv])

    pltpu.sync_copy(b0, g0_hbm.at[pl.ds(base, SPW)])
    pltpu.sync_copy(b1, g1_hbm.at[pl.ds(base, SPW)])
    pltpu.sync_copy(wsb, wsel_hbm.at[pl.ds(base, SPW)])


def _norm_body(a_ref, b_ref, w_ref, o_ref):
    h = a_ref[...] + b_ref[...]
    wsum = jnp.maximum(w_ref[...], 0.0)
    wsum = jnp.where(wsum > 0.0, wsum, 0.001)
    o_ref[...] = h / wsum


def _normalize(g0, g1, wsel):
    return pl.pallas_call(
        _norm_body,
        out_shape=jax.ShapeDtypeStruct((NUPPAD, F), jnp.float32),
    )(g0, g1, wsel)


def kernel(features, nidx_down, weights_down, sel_idx_up):
    f_pad = jnp.pad(features, ((0, NPAD - N), (0, 0)))
    w_pad = jnp.pad(weights_down, ((0, NPAD - N), (0, 0)))
    n_pad = jnp.pad(nidx_down, ((0, NPAD - N), (0, 0)))
    nidx2d = n_pad.reshape(NB, EB)
    w2d = w_pad.reshape(NB, EB)
    selpad = jnp.pad(sel_idx_up[:, 0], (0, NUPPAD - NUP))
    zeros = jnp.zeros((RPS, F), jnp.float32)
    zrow = jnp.zeros((NPAD,), jnp.float32)

    contrib = _build_contrib(w_pad, f_pad)
    p0, p1, wsparts = _scatter_kernel()(contrib, nidx2d, w2d, zeros, zrow)
    g0, g1, wsel = _gather_kernel()(p0, p1, wsparts, selpad)
    out = _normalize(g0, g1, wsel.reshape(NUPPAD, 1))
    return out[:NUP]


# speedup vs baseline: 5.2881x; 1.1716x over previous
import dataclasses
import functools

import jax
import jax.numpy as jnp
from jax import lax
from jax.experimental import pallas as pl
from jax.experimental.pallas import tpu as pltpu
from jax.experimental.pallas import tpu_sc as plsc

N = 10000
K = 32
F = 128
NUP = 2500

NPAD = 10240
EB = 128
NB = NPAD * K // EB
NW = 32
NS = 16
BPW = NB // NW
NUPPAD = 2560
SPW = NUPPAD // NW
L = 16
DUMMY = NUPPAD
ACCR = 2688
RA = ACCR // NS
RO = NUPPAD // NS


@functools.cache
def _mesh():
    return plsc.VectorSubcoreMesh(core_axis_name="c", subcore_axis_name="s")


def _sc_params():
    cp = pltpu.CompilerParams()
    if "needs_layout_passes" in pltpu.CompilerParams.__dataclass_fields__:
        cp = dataclasses.replace(cp, needs_layout_passes=False)
    return cp


@functools.cache
def _scatter_kernel():
    return pl.kernel(
        _scatter_body,
        out_type=[
            jax.ShapeDtypeStruct((NUPPAD, F), jnp.float32),
            jax.ShapeDtypeStruct((NUPPAD, F), jnp.float32),
            jax.ShapeDtypeStruct((2, NS, ACCR), jnp.float32),
            jax.ShapeDtypeStruct((NUPPAD,), jnp.int32),
        ],
        mesh=_mesh(),
        scratch_types=[
            pltpu.VMEM_SHARED((ACCR, F), jnp.float32),
            pltpu.VMEM((NPAD,), jnp.int32),
            pltpu.VMEM((NUPPAD,), jnp.int32),
            pltpu.VMEM((ACCR,), jnp.float32),
            pltpu.VMEM((BPW, EB), jnp.int32),
            pltpu.VMEM((BPW * EB,), jnp.int32),
            pltpu.VMEM((BPW * EB,), jnp.float32),
            pltpu.VMEM((2, EB, F), jnp.float32),
            pltpu.VMEM((BPW, EB), jnp.int32),
            pltpu.VMEM((BPW, EB), jnp.float32),
            pltpu.VMEM((SPW,), jnp.int32),
            pltpu.SemaphoreType.DMA((2,)),
        ],
        compiler_params=_sc_params(),
    )


def _scatter_body(feat_hbm, nidx_hbm, w_hbm, sel_hbm, zeros_hbm,
                  p0_hbm, p1_hbm, wsparts_hbm, smap_hbm,
                  acc, remap, selbuf, wsacc, slot2d, src1d, w1d, fbuf,
                  islab, wslab, sbuf, gsem):
    c = lax.axis_index("c")
    s = lax.axis_index("s")
    wid = s * 2 + c
    base = wid * BPW
    jbase = wid * SPW
    iot = lax.iota(jnp.int32, L)

    pltpu.sync_copy(nidx_hbm.at[pl.ds(base, BPW)], islab)
    pltpu.sync_copy(w_hbm.at[pl.ds(base, BPW)], wslab)
    pltpu.sync_copy(sel_hbm, selbuf)
    pltpu.sync_copy(zeros_hbm, acc.at[pl.ds(s * RA, RA)])

    @pl.loop(0, NPAD // L)
    def _(v):
        remap[pl.ds(v * L, L)] = jnp.full((L,), DUMMY, jnp.int32)

    @pl.loop(0, NUPPAD // L)
    def _(g):
        selv = selbuf[pl.ds(g * L, L)]
        plsc.store_scatter(remap, [selv], g * L + iot)

    @pl.loop(0, ACCR // L)
    def _(v):
        wsacc[pl.ds(v * L, L)] = jnp.zeros((L,), jnp.float32)

    @pl.loop(0, BPW)
    def _(r):
        for v in range(EB // L):
            slot2d[r, pl.ds(v * L, L)] = jnp.full((L,), DUMMY, jnp.int32)
            src1d[pl.ds(r * EB + v * L, L)] = jnp.zeros((L,), jnp.int32)

    @pl.loop(0, SPW // L)
    def _(g):
        selv = selbuf[pl.ds(jbase + g * L, L)]
        sbuf[pl.ds(g * L, L)] = plsc.load_gather(remap, [selv])

    pltpu.sync_copy(sbuf, smap_hbm.at[pl.ds(jbase, SPW)])
    plsc.subcore_barrier()

    def _batch(j, cur):
        for g in range(EB // L):
            dst = islab[j, pl.ds(g * L, L)]
            wv = wslab[j, pl.ds(g * L, L)]
            slotv = plsc.load_gather(remap, [dst])
            plsc.addupdate_scatter(wsacc, [slotv], wv)
            mask = slotv < DUMMY
            pos = cur + plsc.cumsum(mask.astype(jnp.int32)) - 1
            rowv = jax.lax.shift_right_logical(pos, 7)
            colv = pos & (EB - 1)
            srcv = jax.lax.shift_right_logical(
                (base + j) * EB + g * L + iot, 5)
            plsc.store_scatter(slot2d, [rowv, colv], slotv, mask=mask)
            plsc.store_scatter(src1d, [pos], srcv, mask=mask)
            plsc.store_scatter(w1d, [pos], wv, mask=mask)
            cur = cur + plsc.all_reduce_population_count(mask)
        return cur

    cur = lax.fori_loop(0, BPW, _batch, jnp.zeros((L,), jnp.int32))
    ncomp = lax.reduce_max(cur, axes=(0,))
    nch = jax.lax.shift_right_logical(ncomp + (EB - 1), 7)

    def _g_dma(ch, b):
        return pltpu.make_async_copy(
            feat_hbm.at[src1d.at[pl.ds(ch * EB, EB)]], fbuf.at[b],
            gsem.at[b])

    @pl.when(0 < nch)
    def _():
        _g_dma(0, 0).start()

    @pl.when(1 < nch)
    def _():
        _g_dma(1, 1).start()

    @pl.loop(0, BPW, step=2)
    def _(c2):
        for b in range(2):
            ch = c2 + b

            @pl.when(ch < nch)
            def _():
                _g_dma(ch, b).wait()

                @pl.loop(0, EB)
                def _(e):
                    wspl = plsc.load_gather(
                        w1d, [jnp.full((L,), ch * EB + e, jnp.int32)])
                    for v in range(F // L):
                        fbuf[b, e, pl.ds(v * L, L)] = (
                            fbuf[b, e, pl.ds(v * L, L)] * wspl)

                pltpu.sync_copy(fbuf.at[b], acc.at[slot2d.at[ch]], add=True)

                @pl.when(ch + 2 < nch)
                def _():
                    _g_dma(ch + 2, b).start()

    plsc.subcore_barrier()

    @pl.when(c == 0)
    def _():
        pltpu.sync_copy(acc.at[pl.ds(s * RO, RO)],
                        p0_hbm.at[pl.ds(s * RO, RO)])

    @pl.when(c == 1)
    def _():
        pltpu.sync_copy(acc.at[pl.ds(s * RO, RO)],
                        p1_hbm.at[pl.ds(s * RO, RO)])

    pltpu.sync_copy(wsacc, wsparts_hbm.at[c, s])


@functools.cache
def _gather_kernel():
    return pl.kernel(
        _gather_body,
        out_type=[
            jax.ShapeDtypeStruct((NUPPAD, F), jnp.float32),
            jax.ShapeDtypeStruct((NUPPAD, F), jnp.float32),
            jax.ShapeDtypeStruct((NUPPAD,), jnp.float32),
        ],
        mesh=_mesh(),
        scratch_types=[
            pltpu.VMEM((SPW,), jnp.int32),
            pltpu.VMEM((SPW, F), jnp.float32),
            pltpu.VMEM((SPW, F), jnp.float32),
            pltpu.VMEM((2, NS, ACCR), jnp.float32),
            pltpu.VMEM((ACCR,), jnp.float32),
            pltpu.VMEM((SPW,), jnp.float32),
        ],
        compiler_params=_sc_params(),
    )


def _gather_body(p0_hbm, p1_hbm, wsparts_hbm, smap_hbm,
                 g0_hbm, g1_hbm, wsel_hbm, idxb, b0, b1, wall, wt, wsb):
    c = lax.axis_index("c")
    s = lax.axis_index("s")
    base = (s * 2 + c) * SPW
    pltpu.sync_copy(smap_hbm.at[pl.ds(base, SPW)], idxb)
    pltpu.sync_copy(p0_hbm.at[idxb], b0)
    pltpu.sync_copy(p1_hbm.at[idxb], b1)
    pltpu.sync_copy(wsparts_hbm, wall)

    @pl.loop(0, ACCR // L)
    def _(v):
        tot = wall[0, 0, pl.ds(v * L, L)]
        for c2 in range(2):
            for r in range(NS):
                if c2 or r:
                    tot = tot + wall[c2, r, pl.ds(v * L, L)]
        wt[pl.ds(v * L, L)] = tot

    @pl.loop(0, SPW // L)
    def _(g):
        sv = idxb[pl.ds(g * L, L)]
        wsb[pl.ds(g * L, L)] = plsc.load_gather(wt, [sv])

    pltpu.sync_copy(b0, g0_hbm.at[pl.ds(base, SPW)])
    pltpu.sync_copy(b1, g1_hbm.at[pl.ds(base, SPW)])
    pltpu.sync_copy(wsb, wsel_hbm.at[pl.ds(base, SPW)])


def _norm_body(a_ref, b_ref, w_ref, o_ref):
    h = a_ref[...] + b_ref[...]
    wsum = jnp.maximum(w_ref[...], 0.0)
    wsum = jnp.where(wsum > 0.0, wsum, 0.001)
    o_ref[...] = h / wsum


def _normalize(g0, g1, wsel):
    return pl.pallas_call(
        _norm_body,
        out_shape=jax.ShapeDtypeStruct((NUPPAD, F), jnp.float32),
    )(g0, g1, wsel)


def kernel(features, nidx_down, weights_down, sel_idx_up):
    f_pad = jnp.pad(features, ((0, NPAD - N), (0, 0)))
    w_pad = jnp.pad(weights_down, ((0, NPAD - N), (0, 0)))
    n_pad = jnp.pad(nidx_down, ((0, NPAD - N), (0, 0)))
    nidx2d = n_pad.reshape(NB, EB)
    w2d = w_pad.reshape(NB, EB)
    selpad = jnp.pad(sel_idx_up[:, 0], (0, NUPPAD - NUP))
    zeros = jnp.zeros((RA, F), jnp.float32)

    p0, p1, wsparts, smap = _scatter_kernel()(f_pad, nidx2d, w2d,
                                              selpad, zeros)
    g0, g1, wsel = _gather_kernel()(p0, p1, wsparts, smap)
    out = _normalize(g0, g1, wsel.reshape(NUPPAD, 1))
    return out[:NUP]
